# baseline (speedup 1.0000x reference)
import functools

import jax
import jax.numpy as jnp
from jax import lax
from jax.experimental import pallas as pl
from jax.experimental.pallas import tpu as pltpu
from jax.experimental.pallas import tpu_sc as plsc

_LANES = 16


def _router_body(cap, n_experts, x_ref, wr_ref, br_ref, wn_ref, bn_ref,
                 noise_ref, dst1_ref, dst2_ref, g1_ref, g2_ref, xpk_ref,
                 cnt_ref):
    i = pl.program_id(0)

    @pl.when(i == 0)
    def _():
        cnt_ref[...] = jnp.zeros_like(cnt_ref)

    xb = x_ref[...]
    dh = xb.shape[1] // 2
    xpk_ref[...] = pltpu.pack_elementwise(
        [xb[:, :dh], xb[:, dh:]], packed_dtype=jnp.bfloat16)
    logits = jnp.dot(xb, wr_ref[...], preferred_element_type=jnp.float32)
    logits = logits + br_ref[...]
    nlog = jnp.dot(xb, wn_ref[...], preferred_element_type=jnp.float32)
    nlog = nlog + bn_ref[...]
    sp = jnp.maximum(nlog, 0.0) + jnp.log1p(jnp.exp(-jnp.abs(nlog)))
    noisy = logits + noise_ref[...] * sp

    tb = noisy.shape[0]
    iota_e = lax.broadcasted_iota(jnp.int32, noisy.shape, 1)
    m1 = jnp.max(noisy, axis=1, keepdims=True)
    i1 = jnp.min(jnp.where(noisy == m1, iota_e, n_experts), axis=1,
                 keepdims=True)
    masked = jnp.where(iota_e == i1, -jnp.inf, noisy)
    m2 = jnp.max(masked, axis=1, keepdims=True)
    i2 = jnp.min(jnp.where(masked == m2, iota_e, n_experts), axis=1,
                 keepdims=True)

    e2 = jnp.exp(m2 - m1)
    g1_ref[...] = 1.0 / (1.0 + e2)
    g2_ref[...] = e2 / (1.0 + e2)

    mem = ((iota_e == i1) | (iota_e == i2)).astype(jnp.float32)
    r_iota = lax.broadcasted_iota(jnp.int32, (tb, tb), 0)
    c_iota = lax.broadcasted_iota(jnp.int32, (tb, tb), 1)
    tri = (r_iota > c_iota).astype(jnp.float32)
    prefix = jnp.dot(tri, mem, preferred_element_type=jnp.float32)
    ranks = prefix + cnt_ref[...]
    cnt_ref[...] = cnt_ref[...] + jnp.sum(mem, axis=0, keepdims=True)

    r1 = jnp.sum(jnp.where(iota_e == i1, ranks, 0.0), axis=1,
                 keepdims=True).astype(jnp.int32)
    r2 = jnp.sum(jnp.where(iota_e == i2, ranks, 0.0), axis=1,
                 keepdims=True).astype(jnp.int32)
    dst1_ref[...] = jnp.where(r1 < cap, i1 * cap + r1, -1)
    dst2_ref[...] = jnp.where(r2 < cap, i2 * cap + r2, -1)


def _router(x2, wr, br, wn, bn, noise2, cap, interpret=False):
    n, d = x2.shape
    n_experts = wr.shape[1]
    tb = 512
    grid = (n // tb,)
    out = pl.pallas_call(
        functools.partial(_router_body, cap, n_experts),
        grid=grid,
        in_specs=[
            pl.BlockSpec((tb, d), lambda i: (i, 0)),
            pl.BlockSpec((d, n_experts), lambda i: (0, 0)),
            pl.BlockSpec((1, n_experts), lambda i: (0, 0)),
            pl.BlockSpec((d, n_experts), lambda i: (0, 0)),
            pl.BlockSpec((1, n_experts), lambda i: (0, 0)),
            pl.BlockSpec((tb, n_experts), lambda i: (i, 0)),
        ],
        out_specs=[
            pl.BlockSpec((tb, 1), lambda i: (i, 0)),
            pl.BlockSpec((tb, 1), lambda i: (i, 0)),
            pl.BlockSpec((tb, 1), lambda i: (i, 0)),
            pl.BlockSpec((tb, 1), lambda i: (i, 0)),
            pl.BlockSpec((tb, d // 2), lambda i: (i, 0)),
        ],
        out_shape=[
            jax.ShapeDtypeStruct((n, 1), jnp.int32),
            jax.ShapeDtypeStruct((n, 1), jnp.int32),
            jax.ShapeDtypeStruct((n, 1), jnp.float32),
            jax.ShapeDtypeStruct((n, 1), jnp.float32),
            jax.ShapeDtypeStruct((n, d // 2), jnp.int32),
        ],
        scratch_shapes=[pltpu.VMEM((1, n_experts), jnp.float32)],
        interpret=interpret,
    )(x2, wr, br.reshape(1, n_experts), wn, bn.reshape(1, n_experts), noise2)
    return out


def _dispatch(dst1, dst2, x2, n_slots, base):
    n, d = x2.shape
    nw = 32
    spw = n_slots // nw
    rows_chunk = 128
    n_chunks = spw // rows_chunk

    mesh = plsc.VectorSubcoreMesh(core_axis_name="c", subcore_axis_name="s")

    @functools.partial(
        pl.kernel,
        out_type=jax.ShapeDtypeStruct((n_slots, d), x2.dtype),
        mesh=mesh,
        compiler_params=pltpu.CompilerParams(needs_layout_passes=False),
        scratch_types=[
            pltpu.VMEM((n,), jnp.int32),
            pltpu.VMEM((n,), jnp.int32),
            pltpu.VMEM((spw,), jnp.int32),
            pltpu.VMEM((rows_chunk, d), x2.dtype),
            pltpu.SemaphoreType.DMA,
        ],
    )
    def _k(dst1_hbm, dst2_hbm, x_hbm, xe_hbm, d1_v, d2_v, sel_v, rows_v, sem):
        cid = lax.axis_index("c")
        sid = lax.axis_index("s")
        wid = sid * 2 + cid
        w0 = base + wid * spw

        pltpu.sync_copy(dst1_hbm, d1_v)
        pltpu.sync_copy(dst2_hbm, d2_v)

        zero16 = jnp.zeros((_LANES,), jnp.int32)
        for c in range(spw // _LANES):
            sel_v[pl.ds(c * _LANES, _LANES)] = zero16

        idx16 = lax.iota(jnp.int32, _LANES)

        def scan(d_v):
            def body(p, carry):
                dv = d_v[pl.ds(p * _LANES, _LANES)]
                ids = p * _LANES + idx16
                rel = dv - w0
                msk = (rel >= 0) & (rel < spw)
                relc = jnp.clip(rel, 0, spw - 1)
                plsc.store_scatter(sel_v, [relc], ids, mask=msk)
                return carry
            lax.fori_loop(0, n // _LANES, body, 0)

        scan(d1_v)
        scan(d2_v)

        for ch in range(n_chunks):
            pltpu.async_copy(
                x_hbm.at[sel_v.at[pl.ds(ch * rows_chunk, rows_chunk)]],
                rows_v, sem).wait()
            pltpu.sync_copy(
                rows_v,
                xe_hbm.at[pl.ds(wid * spw + ch * rows_chunk, rows_chunk)])

    return _k(dst1, dst2, x2)


def _ffn_body(xe_ref, w1_ref, b1_ref, w2_ref, b2_ref, y_ref):
    xi = xe_ref[...]
    dh = xi.shape[1]
    xa = pltpu.unpack_elementwise(
        xi, index=0, packed_dtype=jnp.bfloat16,
        unpacked_dtype=jnp.float32).astype(jnp.bfloat16)
    xc = pltpu.unpack_elementwise(
        xi, index=1, packed_dtype=jnp.bfloat16,
        unpacked_dtype=jnp.float32).astype(jnp.bfloat16)
    hid = (jnp.dot(xa, w1_ref[0, :dh].astype(jnp.bfloat16),
                   preferred_element_type=jnp.float32)
           + jnp.dot(xc, w1_ref[0, dh:].astype(jnp.bfloat16),
                     preferred_element_type=jnp.float32))
    hid = jnp.maximum(hid + b1_ref[0, 0], 0.0).astype(jnp.bfloat16)
    part = jnp.dot(hid, w2_ref[0].astype(jnp.bfloat16),
                   preferred_element_type=jnp.float32)
    y_ref[...] = part + b2_ref[0, 0]


def _ffn_alias_body(xe_ref, w1_ref, b1_ref, w2_ref, b2_ref, yin_ref, y_ref):
    _ffn_body(xe_ref, w1_ref, b1_ref, w2_ref, b2_ref, y_ref)


def _ffn(xe, w1, b1, w2, b2, e_off, ne, y_total, y_in=None, interpret=False):
    s, dpk = xe.shape
    n_experts, dm, hdim = w1.shape
    cap = s // ne
    rb = 1024
    nr = cap // rb
    in_specs = [
        pl.BlockSpec((rb, dpk), lambda e, r: (e * nr + r, 0)),
        pl.BlockSpec((1, dm, hdim), lambda e, r: (e + e_off, 0, 0)),
        pl.BlockSpec((1, 1, hdim), lambda e, r: (e + e_off, 0, 0)),
        pl.BlockSpec((1, hdim, dm), lambda e, r: (e + e_off, 0, 0)),
        pl.BlockSpec((1, 1, dm), lambda e, r: (e + e_off, 0, 0)),
    ]
    args = [xe, w1, b1.reshape(n_experts, 1, hdim), w2,
            b2.reshape(n_experts, 1, dm)]
    body = _ffn_body
    aliases = {}
    if y_in is not None:
        in_specs.append(pl.BlockSpec(memory_space=pl.ANY))
        args.append(y_in)
        aliases = {5: 0}
        body = _ffn_alias_body
    y = pl.pallas_call(
        body,
        grid=(ne, nr),
        in_specs=in_specs,
        out_specs=pl.BlockSpec((rb, dm),
                               lambda e, r: ((e + e_off) * nr + r, 0)),
        out_shape=jax.ShapeDtypeStruct((y_total, dm), jnp.float32),
        input_output_aliases=aliases,
        interpret=interpret,
    )(*args)
    return y


def _combine(dst1, dst2, g1, g2, y):
    n = dst1.shape[0]
    d = y.shape[1]
    nw = 32
    tpt = n // nw
    ch = 32
    n_chunks = tpt // ch

    mesh = plsc.VectorSubcoreMesh(core_axis_name="c", subcore_axis_name="s")

    @functools.partial(
        pl.kernel,
        out_type=jax.ShapeDtypeStruct((n, d), jnp.float32),
        mesh=mesh,
        compiler_params=pltpu.CompilerParams(needs_layout_passes=False),
        scratch_types=[
            pltpu.VMEM((tpt,), jnp.int32),
            pltpu.VMEM((tpt,), jnp.int32),
            pltpu.VMEM((tpt,), jnp.float32),
            pltpu.VMEM((tpt,), jnp.float32),
            pltpu.VMEM((ch, d), jnp.float32),
            pltpu.VMEM((ch, d), jnp.float32),
            pltpu.VMEM((ch, d), jnp.float32),
            pltpu.SemaphoreType.DMA,
        ],
    )
    def _k(dst1_hbm, dst2_hbm, g1_hbm, g2_hbm, y_hbm, out_hbm,
           i1_v, i2_v, g1_v, g2_v, rows1_v, rows2_v, outb_v, sem):
        cid = lax.axis_index("c")
        sid = lax.axis_index("s")
        wid = sid * 2 + cid
        t0 = wid * tpt

        pltpu.sync_copy(dst1_hbm.at[pl.ds(t0, tpt)], i1_v)
        pltpu.sync_copy(dst2_hbm.at[pl.ds(t0, tpt)], i2_v)
        pltpu.sync_copy(g1_hbm.at[pl.ds(t0, tpt)], g1_v)
        pltpu.sync_copy(g2_hbm.at[pl.ds(t0, tpt)], g2_v)

        def fix(p, carry):
            sl = pl.ds(p * _LANES, _LANES)
            d1 = i1_v[sl]
            d2 = i2_v[sl]
            v1 = d1 >= 0
            v2 = d2 >= 0
            i1_v[sl] = jnp.where(v1, d1, 0)
            i2_v[sl] = jnp.where(v2, d2, 0)
            g1_v[sl] = jnp.where(v1, g1_v[sl], 0.0)
            g2_v[sl] = jnp.where(v2, g2_v[sl], 0.0)
            return carry

        lax.fori_loop(0, tpt // _LANES, fix, 0)

        zeros16 = jnp.zeros((_LANES,), jnp.int32)
        for c in range(n_chunks):
            c0 = c * ch
            pltpu.async_copy(y_hbm.at[i1_v.at[pl.ds(c0, ch)]], rows1_v,
                             sem).wait()
            pltpu.async_copy(y_hbm.at[i2_v.at[pl.ds(c0, ch)]], rows2_v,
                             sem).wait()

            def tok(i, carry):
                g1s = plsc.load_gather(g1_v, [c0 + i + zeros16])
                g2s = plsc.load_gather(g2_v, [c0 + i + zeros16])
                for j in range(d // _LANES):
                    sl = pl.ds(j * _LANES, _LANES)
                    outb_v[i, sl] = (g1s * rows1_v[i, sl]
                                     + g2s * rows2_v[i, sl])
                return carry

            lax.fori_loop(0, ch, tok, 0)
            pltpu.sync_copy(outb_v, out_hbm.at[pl.ds(t0 + c0, ch)])

    return _k(dst1, dst2, g1, g2, y)


def kernel(x, Wr, br, Wn, bn, W1, b1, W2, b2, noise):
    bsz, tlen, d = x.shape
    n_experts = Wr.shape[1]
    top_k = 2
    n = bsz * tlen
    cap = int(n * top_k / n_experts)
    n_slots = n_experts * cap

    x2 = x.reshape(n, d)
    noise2 = noise.reshape(n, n_experts)

    dst1, dst2, g1, g2, xpk = _router(x2, Wr, br, Wn, bn, noise2, cap)
    dst1 = dst1.reshape(n)
    dst2 = dst2.reshape(n)
    g1 = g1.reshape(n)
    g2 = g2.reshape(n)

    half = n_slots // 2
    ne = n_experts // 2
    xe0 = _dispatch(dst1, dst2, xpk, half, 0)
    xe1 = _dispatch(dst1, dst2, xpk, half, half)
    y0 = _ffn(xe0, W1, b1, W2, b2, 0, ne, n_slots)
    y = _ffn(xe1, W1, b1, W2, b2, ne, ne, n_slots, y_in=y0)
    out = _combine(dst1, dst2, g1, g2, y)
    return out.reshape(bsz, tlen, d)


# speedup vs baseline: 1.0565x; 1.0565x over previous
import functools

import jax
import jax.numpy as jnp
from jax import lax
from jax.experimental import pallas as pl
from jax.experimental.pallas import tpu as pltpu
from jax.experimental.pallas import tpu_sc as plsc

_LANES = 16


def _router_body(cap, n_experts, x_ref, wr_ref, br_ref, wn_ref, bn_ref,
                 noise_ref, dst1_ref, dst2_ref, g1_ref, g2_ref, cnt_ref):
    i = pl.program_id(0)

    @pl.when(i == 0)
    def _():
        cnt_ref[...] = jnp.zeros_like(cnt_ref)

    xb = x_ref[...]
    logits = jnp.dot(xb, wr_ref[...], preferred_element_type=jnp.float32)
    logits = logits + br_ref[...]
    nlog = jnp.dot(xb, wn_ref[...], preferred_element_type=jnp.float32)
    nlog = nlog + bn_ref[...]
    sp = jnp.maximum(nlog, 0.0) + jnp.log1p(jnp.exp(-jnp.abs(nlog)))
    noisy = logits + noise_ref[...] * sp

    tb = noisy.shape[0]
    iota_e = lax.broadcasted_iota(jnp.int32, noisy.shape, 1)
    m1 = jnp.max(noisy, axis=1, keepdims=True)
    i1 = jnp.min(jnp.where(noisy == m1, iota_e, n_experts), axis=1,
                 keepdims=True)
    masked = jnp.where(iota_e == i1, -jnp.inf, noisy)
    m2 = jnp.max(masked, axis=1, keepdims=True)
    i2 = jnp.min(jnp.where(masked == m2, iota_e, n_experts), axis=1,
                 keepdims=True)

    e2 = jnp.exp(m2 - m1)
    g1_ref[...] = 1.0 / (1.0 + e2)
    g2_ref[...] = e2 / (1.0 + e2)

    mem = ((iota_e == i1) | (iota_e == i2)).astype(jnp.float32)
    r_iota = lax.broadcasted_iota(jnp.int32, (tb, tb), 0)
    c_iota = lax.broadcasted_iota(jnp.int32, (tb, tb), 1)
    tri = (r_iota > c_iota).astype(jnp.float32)
    prefix = jnp.dot(tri, mem, preferred_element_type=jnp.float32)
    ranks = prefix + cnt_ref[...]
    cnt_ref[...] = cnt_ref[...] + jnp.sum(mem, axis=0, keepdims=True)

    r1 = jnp.sum(jnp.where(iota_e == i1, ranks, 0.0), axis=1,
                 keepdims=True).astype(jnp.int32)
    r2 = jnp.sum(jnp.where(iota_e == i2, ranks, 0.0), axis=1,
                 keepdims=True).astype(jnp.int32)
    dst1_ref[...] = jnp.where(r1 < cap, i1 * cap + r1, -1)
    dst2_ref[...] = jnp.where(r2 < cap, i2 * cap + r2, -1)


def _router(x2, wr, br, wn, bn, noise2, cap, interpret=False):
    n, d = x2.shape
    n_experts = wr.shape[1]
    tb = 512
    grid = (n // tb,)
    out = pl.pallas_call(
        functools.partial(_router_body, cap, n_experts),
        grid=grid,
        in_specs=[
            pl.BlockSpec((tb, d), lambda i: (i, 0)),
            pl.BlockSpec((d, n_experts), lambda i: (0, 0)),
            pl.BlockSpec((1, n_experts), lambda i: (0, 0)),
            pl.BlockSpec((d, n_experts), lambda i: (0, 0)),
            pl.BlockSpec((1, n_experts), lambda i: (0, 0)),
            pl.BlockSpec((tb, n_experts), lambda i: (i, 0)),
        ],
        out_specs=[
            pl.BlockSpec((tb, 1), lambda i: (i, 0)),
            pl.BlockSpec((tb, 1), lambda i: (i, 0)),
            pl.BlockSpec((tb, 1), lambda i: (i, 0)),
            pl.BlockSpec((tb, 1), lambda i: (i, 0)),
        ],
        out_shape=[
            jax.ShapeDtypeStruct((n, 1), jnp.int32),
            jax.ShapeDtypeStruct((n, 1), jnp.int32),
            jax.ShapeDtypeStruct((n, 1), jnp.float32),
            jax.ShapeDtypeStruct((n, 1), jnp.float32),
        ],
        scratch_shapes=[pltpu.VMEM((1, n_experts), jnp.float32)],
        interpret=interpret,
    )(x2, wr, br.reshape(1, n_experts), wn, bn.reshape(1, n_experts), noise2)
    return out


def _dispatch(dst1, dst2, x2, n_slots, base):
    n, d = x2.shape
    nw = 32
    spw = n_slots // nw
    rows_chunk = 128
    n_chunks = spw // rows_chunk

    mesh = plsc.VectorSubcoreMesh(core_axis_name="c", subcore_axis_name="s")

    @functools.partial(
        pl.kernel,
        out_type=jax.ShapeDtypeStruct((n_slots, d), x2.dtype),
        mesh=mesh,
        compiler_params=pltpu.CompilerParams(needs_layout_passes=False),
        scratch_types=[
            pltpu.VMEM((n,), jnp.int32),
            pltpu.VMEM((n,), jnp.int32),
            pltpu.VMEM((spw,), jnp.int32),
            pltpu.VMEM((rows_chunk, d), x2.dtype),
            pltpu.SemaphoreType.DMA,
        ],
    )
    def _k(dst1_hbm, dst2_hbm, x_hbm, xe_hbm, d1_v, d2_v, sel_v, rows_v, sem):
        cid = lax.axis_index("c")
        sid = lax.axis_index("s")
        wid = sid * 2 + cid
        w0 = base + wid * spw

        pltpu.sync_copy(dst1_hbm, d1_v)
        pltpu.sync_copy(dst2_hbm, d2_v)

        zero16 = jnp.zeros((_LANES,), jnp.int32)
        for c in range(spw // _LANES):
            sel_v[pl.ds(c * _LANES, _LANES)] = zero16

        idx16 = lax.iota(jnp.int32, _LANES)

        def scan(d_v):
            def body(p, carry):
                dv = d_v[pl.ds(p * _LANES, _LANES)]
                ids = p * _LANES + idx16
                rel = dv - w0
                msk = (rel >= 0) & (rel < spw)
                relc = jnp.clip(rel, 0, spw - 1)
                plsc.store_scatter(sel_v, [relc], ids, mask=msk)
                return carry
            lax.fori_loop(0, n // _LANES, body, 0)

        scan(d1_v)
        scan(d2_v)

        for ch in range(n_chunks):
            pltpu.async_copy(
                x_hbm.at[sel_v.at[pl.ds(ch * rows_chunk, rows_chunk)]],
                rows_v, sem).wait()
            pltpu.sync_copy(
                rows_v,
                xe_hbm.at[pl.ds(wid * spw + ch * rows_chunk, rows_chunk)])

    return _k(dst1, dst2, x2)


def _ffn_body(xe_ref, w1_ref, b1_ref, w2_ref, b2_ref, y_ref):
    xb = xe_ref[...].astype(jnp.bfloat16)
    hid = jnp.dot(xb, w1_ref[0].astype(jnp.bfloat16),
                  preferred_element_type=jnp.float32)
    hid = jnp.maximum(hid + b1_ref[0, 0], 0.0).astype(jnp.bfloat16)
    part = jnp.dot(hid, w2_ref[0].astype(jnp.bfloat16),
                   preferred_element_type=jnp.float32)
    y_ref[...] = part + b2_ref[0, 0]


def _ffn_alias_body(xe_ref, w1_ref, b1_ref, w2_ref, b2_ref, yin_ref, y_ref):
    _ffn_body(xe_ref, w1_ref, b1_ref, w2_ref, b2_ref, y_ref)


def _ffn(xe, w1, b1, w2, b2, e_off, ne, y_total, y_in=None, interpret=False):
    s, dpk = xe.shape
    n_experts, dm, hdim = w1.shape
    cap = s // ne
    rb = 1024
    nr = cap // rb
    in_specs = [
        pl.BlockSpec((rb, dpk), lambda e, r: (e * nr + r, 0)),
        pl.BlockSpec((1, dm, hdim), lambda e, r: (e + e_off, 0, 0)),
        pl.BlockSpec((1, 1, hdim), lambda e, r: (e + e_off, 0, 0)),
        pl.BlockSpec((1, hdim, dm), lambda e, r: (e + e_off, 0, 0)),
        pl.BlockSpec((1, 1, dm), lambda e, r: (e + e_off, 0, 0)),
    ]
    args = [xe, w1, b1.reshape(n_experts, 1, hdim), w2,
            b2.reshape(n_experts, 1, dm)]
    body = _ffn_body
    aliases = {}
    if y_in is not None:
        in_specs.append(pl.BlockSpec(memory_space=pl.ANY))
        args.append(y_in)
        aliases = {5: 0}
        body = _ffn_alias_body
    y = pl.pallas_call(
        body,
        grid=(ne, nr),
        in_specs=in_specs,
        out_specs=pl.BlockSpec((rb, dm),
                               lambda e, r: ((e + e_off) * nr + r, 0)),
        out_shape=jax.ShapeDtypeStruct((y_total, dm), jnp.float32),
        input_output_aliases=aliases,
        interpret=interpret,
    )(*args)
    return y


def _combine(dst1, dst2, g1, g2, y):
    n = dst1.shape[0]
    d = y.shape[1]
    nw = 32
    tpt = n // nw
    ch = 16
    n_chunks = tpt // ch

    mesh = plsc.VectorSubcoreMesh(core_axis_name="c", subcore_axis_name="s")

    @functools.partial(
        pl.kernel,
        out_type=jax.ShapeDtypeStruct((n, d), jnp.float32),
        mesh=mesh,
        compiler_params=pltpu.CompilerParams(needs_layout_passes=False),
        scratch_types=[
            pltpu.VMEM((tpt,), jnp.int32),
            pltpu.VMEM((tpt,), jnp.int32),
            pltpu.VMEM((tpt,), jnp.float32),
            pltpu.VMEM((tpt,), jnp.float32),
            pltpu.VMEM((2, ch, d), jnp.float32),
            pltpu.VMEM((2, ch, d), jnp.float32),
            pltpu.VMEM((2, ch, d), jnp.float32),
            pltpu.SemaphoreType.DMA,
            pltpu.SemaphoreType.DMA,
            pltpu.SemaphoreType.DMA,
        ],
    )
    def _k(dst1_hbm, dst2_hbm, g1_hbm, g2_hbm, y_hbm, out_hbm,
           i1_v, i2_v, g1_v, g2_v, rows1_v, rows2_v, outb_v,
           gsem0, gsem1, ssem):
        cid = lax.axis_index("c")
        sid = lax.axis_index("s")
        wid = sid * 2 + cid
        t0 = wid * tpt

        pltpu.sync_copy(dst1_hbm.at[pl.ds(t0, tpt)], i1_v)
        pltpu.sync_copy(dst2_hbm.at[pl.ds(t0, tpt)], i2_v)
        pltpu.sync_copy(g1_hbm.at[pl.ds(t0, tpt)], g1_v)
        pltpu.sync_copy(g2_hbm.at[pl.ds(t0, tpt)], g2_v)

        def fix(p, carry):
            sl = pl.ds(p * _LANES, _LANES)
            d1 = i1_v[sl]
            d2 = i2_v[sl]
            v1 = d1 >= 0
            v2 = d2 >= 0
            i1_v[sl] = jnp.where(v1, d1, 0)
            i2_v[sl] = jnp.where(v2, d2, 0)
            g1_v[sl] = jnp.where(v1, g1_v[sl], 0.0)
            g2_v[sl] = jnp.where(v2, g2_v[sl], 0.0)
            return carry

        lax.fori_loop(0, tpt // _LANES, fix, 0)

        gsems = (gsem0, gsem1)

        def fire(c, buf):
            c0 = c * ch
            d1 = pltpu.async_copy(y_hbm.at[i1_v.at[pl.ds(c0, ch)]],
                                  rows1_v.at[buf], gsems[buf])
            d2 = pltpu.async_copy(y_hbm.at[i2_v.at[pl.ds(c0, ch)]],
                                  rows2_v.at[buf], gsems[buf])
            return d1, d2

        zeros16 = jnp.zeros((_LANES,), jnp.int32)
        pend = [None, None]
        store_pend = [None, None]
        pend[0] = fire(0, 0)
        for c in range(n_chunks):
            buf = c & 1
            obuf = 1 - buf
            if c + 1 < n_chunks:
                if store_pend[obuf] is not None:
                    store_pend[obuf].wait()
                    store_pend[obuf] = None
                pend[obuf] = fire(c + 1, obuf)
            pend[buf][0].wait()
            pend[buf][1].wait()
            if store_pend[buf] is not None:
                store_pend[buf].wait()
                store_pend[buf] = None

            c0 = c * ch

            def tok(i, carry):
                g1s = plsc.load_gather(g1_v, [c0 + i + zeros16])
                g2s = plsc.load_gather(g2_v, [c0 + i + zeros16])
                for j in range(d // _LANES):
                    sl = pl.ds(j * _LANES, _LANES)
                    outb_v[buf, i, sl] = (g1s * rows1_v[buf, i, sl]
                                          + g2s * rows2_v[buf, i, sl])
                return carry

            lax.fori_loop(0, ch, tok, 0)
            store_pend[buf] = pltpu.async_copy(
                outb_v.at[buf], out_hbm.at[pl.ds(t0 + c0, ch)], ssem)
        for sp in store_pend:
            if sp is not None:
                sp.wait()

    return _k(dst1, dst2, g1, g2, y)


def kernel(x, Wr, br, Wn, bn, W1, b1, W2, b2, noise):
    bsz, tlen, d = x.shape
    n_experts = Wr.shape[1]
    top_k = 2
    n = bsz * tlen
    cap = int(n * top_k / n_experts)
    n_slots = n_experts * cap

    x2 = x.reshape(n, d)
    noise2 = noise.reshape(n, n_experts)

    dst1, dst2, g1, g2 = _router(x2, Wr, br, Wn, bn, noise2, cap)
    dst1 = dst1.reshape(n)
    dst2 = dst2.reshape(n)
    g1 = g1.reshape(n)
    g2 = g2.reshape(n)

    half = n_slots // 2
    ne = n_experts // 2
    xe0 = _dispatch(dst1, dst2, x2, half, 0)
    xe1 = _dispatch(dst1, dst2, x2, half, half)
    y0 = _ffn(xe0, W1, b1, W2, b2, 0, ne, n_slots)
    y = _ffn(xe1, W1, b1, W2, b2, ne, ne, n_slots, y_in=y0)
    out = _combine(dst1, dst2, g1, g2, y)
    return out.reshape(bsz, tlen, d)
